# Initial kernel scaffold; baseline (speedup 1.0000x reference)
import jax
import jax.numpy as jnp
from jax.experimental import pallas as pl


def kernel(H, mask):
    raise NotImplementedError("write your pallas kernel here")


# speedup vs baseline: 1.2708x; 1.2708x over previous
import functools

import jax
import jax.numpy as jnp
from jax import lax
from jax.experimental import pallas as pl
from jax.experimental.pallas import tpu as pltpu

_Q = 0.15
_INF_KEY = 2139095041


def _score_select_kernel(h_ref, m_ref, sel_ref, kf_ref, keys_s, *, nb, nt, tb, t):
    b = pl.program_id(0)
    ti = pl.program_id(1)
    x = h_ref[0]
    sc = jnp.sqrt(jnp.sum(x * x, axis=1))
    key = jnp.where(m_ref[0, 0] != 0,
                    lax.bitcast_convert_type(sc, jnp.int32),
                    jnp.int32(-1))
    keys_s[pl.ds(b, 1), pl.ds(ti * tb, tb)] = key.reshape(1, tb)

    @pl.when((b == nb - 1) & (ti == nt - 1))
    def _finalize():
        keys = keys_s[...]
        validf = jnp.sum((keys >= 0).astype(jnp.float32), axis=1, keepdims=True)
        kf = jnp.max(jnp.maximum(jnp.ceil(jnp.float32(_Q) * validf), 1.0))
        ki = kf.astype(jnp.int32)

        def bs_key(_, carry):
            lo, hi = carry
            mid = lo + (hi - lo) // 2
            cnt = jnp.sum((keys >= mid).astype(jnp.int32), axis=1, keepdims=True)
            ge = cnt >= ki
            return jnp.where(ge, mid, lo), jnp.where(ge, hi, mid)

        lo0 = jnp.full((nb, 1), -1, jnp.int32)
        hi0 = jnp.full((nb, 1), _INF_KEY, jnp.int32)
        thr, _ = lax.fori_loop(0, 31, bs_key, (lo0, hi0))

        c1 = jnp.sum((keys > thr).astype(jnp.int32), axis=1, keepdims=True)
        r = ki - c1
        tie = keys == thr
        iot = lax.broadcasted_iota(jnp.int32, (1, t), 1)

        def bs_idx(_, carry):
            lo, hi = carry
            mid = (lo + hi) // 2
            c = jnp.sum((tie & (iot < mid)).astype(jnp.int32), axis=1,
                        keepdims=True)
            ge = c >= r
            return jnp.where(ge, lo, mid + 1), jnp.where(ge, mid, hi)

        lo1 = jnp.zeros((nb, 1), jnp.int32)
        hi1 = jnp.full((nb, 1), t, jnp.int32)
        _, ibound = lax.fori_loop(0, 13, bs_idx, (lo1, hi1))

        sel = (keys > thr) | (tie & (iot < ibound))
        sel_ref[...] = sel.astype(jnp.int32).reshape(nb, 1, t)
        kf_ref[0, 0] = kf


def _pool_kernel(h_ref, sel_ref, kf_ref, out_ref, acc, *, nt, tb):
    ti = pl.program_id(1)

    @pl.when(ti == 0)
    def _init():
        acc[...] = jnp.zeros_like(acc)

    x = h_ref[0]
    s = sel_ref[0, 0].astype(jnp.float32).reshape(1, tb)
    acc[...] += jnp.dot(s, x, preferred_element_type=jnp.float32)

    @pl.when(ti == nt - 1)
    def _write():
        out_ref[0] = acc[...] / kf_ref[0, 0]


def kernel(H, mask):
    B, T, D = H.shape
    tb = 512
    nt = T // tb
    m3 = mask.astype(jnp.int32).reshape(B, 1, T)

    sel3, kf = pl.pallas_call(
        functools.partial(_score_select_kernel, nb=B, nt=nt, tb=tb, t=T),
        grid=(B, nt),
        in_specs=[
            pl.BlockSpec((1, tb, D), lambda b, ti: (b, ti, 0)),
            pl.BlockSpec((1, 1, tb), lambda b, ti: (b, 0, ti)),
        ],
        out_specs=[
            pl.BlockSpec((B, 1, T), lambda b, ti: (0, 0, 0)),
            pl.BlockSpec(memory_space=pltpu.SMEM, block_shape=(1, 1),
                         index_map=lambda b, ti: (0, 0)),
        ],
        out_shape=[
            jax.ShapeDtypeStruct((B, 1, T), jnp.int32),
            jax.ShapeDtypeStruct((1, 1), jnp.float32),
        ],
        scratch_shapes=[pltpu.VMEM((B, T), jnp.int32)],
    )(H, m3)

    out3 = pl.pallas_call(
        functools.partial(_pool_kernel, nt=nt, tb=tb),
        grid=(B, nt),
        in_specs=[
            pl.BlockSpec((1, tb, D), lambda b, ti: (b, ti, 0)),
            pl.BlockSpec((1, 1, tb), lambda b, ti: (b, 0, ti)),
            pl.BlockSpec(memory_space=pltpu.SMEM, block_shape=(1, 1),
                         index_map=lambda b, ti: (0, 0)),
        ],
        out_specs=pl.BlockSpec((1, 1, D), lambda b, ti: (b, 0, 0)),
        out_shape=jax.ShapeDtypeStruct((B, 1, D), jnp.float32),
        scratch_shapes=[pltpu.VMEM((1, D), jnp.float32)],
    )(H, sel3, kf)

    return out3.reshape(B, D)
